# Initial kernel scaffold; baseline (speedup 1.0000x reference)
import jax
import jax.numpy as jnp
from jax.experimental import pallas as pl


def kernel(input_xyz, coord_xyz, input_f, coord_f, Wf, bf, Ws, bs, gamma, beta, Wh, bh, Wo, bo):
    raise NotImplementedError("write your pallas kernel here")


# speedup vs baseline: 7.0627x; 7.0627x over previous
import functools

import jax
import jax.numpy as jnp
from jax.experimental import pallas as pl
from jax.experimental.pallas import tpu as pltpu

_B, _NP, _DIM, _K = 4, 1024, 128, 16
_HID, _NPROP = 64, 5
_NBLK = 256
_HI = jax.lax.Precision.HIGHEST


def _sigmoid(u):
    return 1.0 / (1.0 + jnp.exp(-u))


def _softplus(v):
    return jnp.maximum(v, 0.0) + jnp.log1p(jnp.exp(-jnp.abs(v)))


def _knn_cgconv_body(xyz_ref, xyzt_ref, x_ref, wf_ref, bf_ref, ws_ref, bs_ref,
                     agg_ref, sums_ref, p_ref, r_ref, qsh_ref, qsl_ref):
    g = pl.program_id(0)
    b_in_graph = jax.lax.rem(g, _B)

    x = x_ref[0].astype(jnp.bfloat16)
    wf = wf_ref[...].astype(jnp.bfloat16)
    ws = ws_ref[...].astype(jnp.bfloat16)
    dot = functools.partial(jnp.dot, preferred_element_type=jnp.float32)
    p_ref[...] = dot(x, wf[:_DIM]) + bf_ref[...]
    r_ref[...] = dot(x, ws[:_DIM]) + bs_ref[...]
    q = dot(x, wf[_DIM:])
    s = dot(x, ws[_DIM:])
    qs = jnp.concatenate([q, s], axis=1)
    hi = qs.astype(jnp.bfloat16)
    qsh_ref[...] = hi
    qsl_ref[...] = (qs - hi.astype(jnp.float32)).astype(jnp.bfloat16)

    total_sum = jnp.zeros((1, _DIM), jnp.float32)
    total_sq = jnp.zeros((1, _DIM), jnp.float32)

    for nb in range(_NP // _NBLK):
        base = nb * _NBLK
        lane = jax.lax.broadcasted_iota(jnp.int32, (_NBLK, _NP), 1)
        rows = base + jax.lax.broadcasted_iota(jnp.int32, (_NBLK, _NP), 0)
        d2 = jnp.zeros((_NBLK, _NP), jnp.float32)
        for c in range(3):
            col = xyz_ref[0, pl.ds(base, _NBLK), c:c + 1]
            row = xyzt_ref[0, c:c + 1, :]
            t = col - row
            d2 = d2 + t * t
        d2 = d2 + jnp.where(lane == rows, 1e10, 0.0)

        p_blk = p_ref[pl.ds(base, _NBLK), :]
        r_blk = r_ref[pl.ds(base, _NBLK), :]
        qsh = qsh_ref[...]
        qsl = qsl_ref[...]
        acc = None
        for k in range(_K):
            m = jnp.min(d2, axis=1, keepdims=True)
            cur = jnp.min(jnp.where(d2 == m, lane, jnp.int32(2**30)),
                          axis=1, keepdims=True)
            hit = lane == cur
            d2 = jnp.where(hit, 3e30, d2)
            oh = jnp.where(hit, 1.0, 0.0).astype(jnp.bfloat16)
            gh = jax.lax.dot_general(oh, qsh, (((1,), (0,)), ((), ())),
                                     preferred_element_type=jnp.float32)
            gl = jax.lax.dot_general(oh, qsl, (((1,), (0,)), ((), ())),
                                     preferred_element_type=jnp.float32)
            gat = gh + gl
            u = p_blk + gat[:, :_DIM]
            v = r_blk + gat[:, _DIM:]
            msg = _sigmoid(u) * _softplus(v)
            acc = msg if acc is None else jnp.maximum(acc, msg)
        agg_ref[0, pl.ds(base, _NBLK), :] = acc
        total_sum = total_sum + jnp.sum(acc, axis=0, keepdims=True)
        total_sq = total_sq + jnp.sum(acc * acc, axis=0, keepdims=True)

    local = jnp.concatenate([total_sum[:, None, :], total_sq[:, None, :]],
                            axis=1)

    @pl.when(b_in_graph == 0)
    def _():
        sums_ref[...] = local

    @pl.when(b_in_graph != 0)
    def _():
        sums_ref[...] = sums_ref[...] + local


def _cross_prop_body(x0_ref, x1_ref, agg0_ref, agg1_ref, sums_ref,
                     gamma_ref, beta_ref, wh_ref, bh_ref, wo_ref, bo_ref,
                     out0_ref, out1_ref, f0_ref, f1_ref, h0_ref, h1_ref):
    gamma = gamma_ref[...]
    beta = beta_ref[...]
    n_nodes = jnp.float32(_B * _NP)

    for gi, (x_ref, agg_ref, f_ref) in enumerate(
            ((x0_ref, agg0_ref, f0_ref), (x1_ref, agg1_ref, f1_ref))):
        ssum = sums_ref[gi, 0:1, :]
        ssq = sums_ref[gi, 1:2, :]
        mu = ssum / n_nodes
        var = ssq / n_nodes - mu * mu
        inv = jax.lax.rsqrt(var + 1e-5)
        f_ref[...] = x_ref[0] + (agg_ref[0] - mu) * inv * gamma + beta

    def rowsoftmax(t):
        m = jnp.max(t, axis=1, keepdims=True)
        e = jnp.exp(t - m)
        return e / jnp.sum(e, axis=1, keepdims=True)

    bf16 = jnp.bfloat16
    dot = functools.partial(jnp.dot, preferred_element_type=jnp.float32)
    for l in range(_NPROP):
        wh = wh_ref[l].astype(bf16)
        bh = bh_ref[l]
        wo = wo_ref[l].astype(bf16)
        bo = bo_ref[l]
        f0 = f0_ref[...].astype(bf16)
        f1 = f1_ref[...].astype(bf16)
        for nb in range(_NP // _NBLK):
            base = nb * _NBLK
            f0b = f0[base:base + _NBLK, :]
            f1b = f1[base:base + _NBLK, :]
            s = jax.lax.dot_general(f0b, f1, (((1,), (1,)), ((), ())),
                                    preferred_element_type=jnp.float32)
            st = jax.lax.dot_general(f1b, f0, (((1,), (1,)), ((), ())),
                                     preferred_element_type=jnp.float32)
            a0 = rowsoftmax(s).astype(bf16)
            a1t = rowsoftmax(st).astype(bf16)
            att0 = dot(a0, f1)
            att1 = dot(a1t, f0)
            mu0 = f0_ref[pl.ds(base, _NBLK), :] - att0
            mu1 = f1_ref[pl.ds(base, _NBLK), :] - att1
            h0 = dot(jax.nn.relu(
                dot(f0b, wh[:_DIM])
                + dot(mu0.astype(bf16), wh[_DIM:]) + bh).astype(bf16),
                wo) + bo
            h1 = dot(jax.nn.relu(
                dot(f1b, wh[:_DIM])
                + dot(mu1.astype(bf16), wh[_DIM:]) + bh).astype(bf16),
                wo) + bo
            h0_ref[pl.ds(base, _NBLK), :] = h0
            h1_ref[pl.ds(base, _NBLK), :] = h1
        f0_ref[...] = f0_ref[...] + h0_ref[...]
        f1_ref[...] = f1_ref[...] + h1_ref[...]

    out0_ref[0] = f0_ref[...]
    out1_ref[0] = f1_ref[...]


def kernel(input_xyz, coord_xyz, input_f, coord_f, Wf, bf, Ws, bs, gamma,
           beta, Wh, bh, Wo, bo):
    f32 = jnp.float32
    xyz = jnp.concatenate([input_xyz, coord_xyz], axis=0)
    xyzt = jnp.pad(jnp.transpose(xyz, (0, 2, 1)),
                   ((0, 0), (0, 5), (0, 0)))
    x_s = jnp.concatenate([input_f, coord_f], axis=0)

    bcast = lambda shape: pl.BlockSpec(shape, lambda g: (0,) * len(shape))
    per_g = lambda shape: pl.BlockSpec(shape, lambda g: (g,) + (0,) * (len(shape) - 1))

    agg, sums = pl.pallas_call(
        _knn_cgconv_body,
        grid=(2 * _B,),
        in_specs=[
            per_g((1, _NP, 3)),
            per_g((1, 8, _NP)),
            per_g((1, _NP, _DIM)),
            bcast((2 * _DIM, _DIM)),
            bcast((1, _DIM)),
            bcast((2 * _DIM, _DIM)),
            bcast((1, _DIM)),
        ],
        out_specs=[
            per_g((1, _NP, _DIM)),
            pl.BlockSpec((1, 2, _DIM), lambda g: (g // _B, 0, 0)),
        ],
        out_shape=[
            jax.ShapeDtypeStruct((2 * _B, _NP, _DIM), f32),
            jax.ShapeDtypeStruct((2, 2, _DIM), f32),
        ],
        scratch_shapes=[
            pltpu.VMEM((_NP, _DIM), f32),
            pltpu.VMEM((_NP, _DIM), f32),
            pltpu.VMEM((_NP, 2 * _DIM), jnp.bfloat16),
            pltpu.VMEM((_NP, 2 * _DIM), jnp.bfloat16),
        ],
    )(xyz, xyzt, x_s, Wf, bf.reshape(1, _DIM), Ws, bs.reshape(1, _DIM))

    out0, out1 = pl.pallas_call(
        _cross_prop_body,
        grid=(_B,),
        in_specs=[
            per_g((1, _NP, _DIM)),
            per_g((1, _NP, _DIM)),
            pl.BlockSpec((1, _NP, _DIM), lambda b: (b, 0, 0)),
            pl.BlockSpec((1, _NP, _DIM), lambda b: (b + _B, 0, 0)),
            bcast((2, 2, _DIM)),
            bcast((1, _DIM)),
            bcast((1, _DIM)),
            bcast((_NPROP, 2 * _DIM, _HID)),
            bcast((_NPROP, 1, _HID)),
            bcast((_NPROP, _HID, _DIM)),
            bcast((_NPROP, 1, _DIM)),
        ],
        out_specs=[
            per_g((1, _NP, _DIM)),
            per_g((1, _NP, _DIM)),
        ],
        out_shape=[
            jax.ShapeDtypeStruct((_B, _NP, _DIM), f32),
            jax.ShapeDtypeStruct((_B, _NP, _DIM), f32),
        ],
        scratch_shapes=[
            pltpu.VMEM((_NP, _DIM), f32),
            pltpu.VMEM((_NP, _DIM), f32),
            pltpu.VMEM((_NP, _DIM), f32),
            pltpu.VMEM((_NP, _DIM), f32),
        ],
    )(input_f, coord_f, agg, agg, sums, gamma.reshape(1, _DIM),
      beta.reshape(1, _DIM), Wh, bh.reshape(_NPROP, 1, _HID), Wo,
      bo.reshape(_NPROP, 1, _DIM))

    return (out0.reshape(-1, _DIM), out1.reshape(-1, _DIM))
